# Initial kernel scaffold; baseline (speedup 1.0000x reference)
import jax
import jax.numpy as jnp
from jax.experimental import pallas as pl


def kernel(x, coords, tables, freq_bands):
    raise NotImplementedError("write your pallas kernel here")


# speedup vs baseline: 8.7832x; 8.7832x over previous
import jax
import jax.numpy as jnp
from jax import lax
from jax.experimental import pallas as pl
from jax.experimental.pallas import tpu as pltpu
from jax.experimental.pallas import tpu_sc as plsc

NUM_LEVELS = 16
LEVEL_DIM = 2
TABLE_SIZE = 65536
D = NUM_LEVELS * LEVEL_DIM
PRIMES = (1.0, 2654435761.0, 805459861.0, 3674653429.0)

NW = 32
CP = 512
GROUPS = CP // 16
IDXN = CP * NUM_LEVELS
GSUB = IDXN // 128
L = 16


def _hash_level(c0, c1, c2, c3, f):
    t0 = c0 * f
    h = t0.astype(jnp.int32)
    for c, p in ((c1, PRIMES[1]), (c2, PRIMES[2]), (c3, PRIMES[3])):
        v = (c * f) * p
        k = (v * (2.0 ** -16)).astype(jnp.int32)
        r = v - k.astype(jnp.float32) * 65536.0
        h = h ^ r.astype(jnp.int32)
    return h


def _body(x_hbm, coords_hbm, tbl_hbm, freq_hbm, out_hbm,
          freq_v, coords_v, idx_v, rows_v, out_v, sem_in, sem_g):
    wid = lax.axis_index("s") * 2 + lax.axis_index("c")
    pltpu.sync_copy(freq_hbm, freq_v)

    lanes = lax.iota(jnp.int32, L)
    iota4 = lanes * 4
    rowpat = lanes // 2
    colpat = lanes % 2
    addpat = rowpat * D + colpat
    freq_vec = freq_v[...]
    fs = [freq_vec.at[jnp.full_like(lanes, i)].get(mode="promise_in_bounds")
          for i in range(NUM_LEVELS)]

    ppw = (8 * 65536) // NW
    base_pt = wid * ppw

    def chunk_body(c, _):
        row0 = base_pt + c * CP
        pltpu.sync_copy(x_hbm.at[pl.ds(row0 * D, CP * D)], out_v)
        pltpu.sync_copy(coords_hbm.at[pl.ds(row0 * 4, CP * 4)], coords_v)

        def group_body(g, _):
            cb = g * 64
            cs = [plsc.load_gather(coords_v, [cb + iota4 + j])
                  for j in range(4)]
            cs = [jnp.minimum(jnp.maximum(cj, 0.0), 1.0) for cj in cs]
            for i in range(NUM_LEVELS):
                h = _hash_level(*cs, fs[i])
                idx_v[pl.ds(i * CP + g * 16, 16)] = h + i * TABLE_SIZE
            return 0

        lax.fori_loop(0, GROUPS, group_body, 0, unroll=False)

        def fire(s, _):
            pltpu.async_copy(tbl_hbm.at[idx_v.at[pl.ds(s * 128, 128)]],
                             rows_v.at[pl.ds(s * 128, 128)], sem_g)
            return 0

        lax.fori_loop(0, GSUB, fire, 0, unroll=False)

        def drain(s, _):
            pltpu.make_async_copy(tbl_hbm.at[idx_v.at[pl.ds(s * 128, 128)]],
                                  rows_v.at[pl.ds(s * 128, 128)], sem_g).wait()
            return 0

        lax.fori_loop(0, GSUB, drain, 0, unroll=False)

        def add_body(g8, _):
            for i in range(NUM_LEVELS):
                val = plsc.load_gather(
                    rows_v, [i * CP + g8 * 8 + rowpat, colpat])
                plsc.addupdate_scatter(
                    out_v, [g8 * (8 * D) + 2 * i + addpat], val)
            return 0

        lax.fori_loop(0, CP // 8, add_body, 0, unroll=False)

        pltpu.sync_copy(out_v, out_hbm.at[pl.ds(row0 * D, CP * D)])
        return 0

    lax.fori_loop(0, ppw // CP, chunk_body, 0, unroll=False)


def kernel(x, coords, tables, freq_bands):
    B, N, Dm = x.shape
    P = B * N
    xf = x.reshape(P * D)
    cf = coords.reshape(P * 4)
    tbl = jnp.pad(tables.reshape(NUM_LEVELS * TABLE_SIZE, LEVEL_DIM),
                  ((0, 0), (0, 8 - LEVEL_DIM)))

    mesh = plsc.VectorSubcoreMesh(core_axis_name="c", subcore_axis_name="s")
    run = pl.kernel(
        _body,
        mesh=mesh,
        compiler_params=pltpu.CompilerParams(needs_layout_passes=False,
                                             use_tc_tiling_on_sc=False),
        out_type=jax.ShapeDtypeStruct((P * D,), jnp.float32),
        scratch_types=[
            pltpu.VMEM((NUM_LEVELS,), jnp.float32),
            pltpu.VMEM((CP * 4,), jnp.float32),
            pltpu.VMEM((IDXN,), jnp.int32),
            pltpu.VMEM((IDXN, 8), jnp.float32),
            pltpu.VMEM((CP * D,), jnp.float32),
            pltpu.SemaphoreType.DMA,
            pltpu.SemaphoreType.DMA,
        ],
    )
    out = run(xf, cf, tbl, freq_bands)
    return out.reshape(B, N, Dm)


# speedup vs baseline: 8.9997x; 1.0247x over previous
import jax
import jax.numpy as jnp
from jax import lax
from jax.experimental import pallas as pl
from jax.experimental.pallas import tpu as pltpu
from jax.experimental.pallas import tpu_sc as plsc

NUM_LEVELS = 16
LEVEL_DIM = 2
TABLE_SIZE = 65536
D = NUM_LEVELS * LEVEL_DIM
PRIMES = (1.0, 2654435761.0, 805459861.0, 3674653429.0)

NW = 32
CP = 512
GROUPS = CP // 16
IDXN = CP * NUM_LEVELS
GSUB = IDXN // 128
L = 16


def _hash_level(c0, c1, c2, c3, f):
    t0 = c0 * f
    h = t0.astype(jnp.int32)
    for c, p in ((c1, PRIMES[1]), (c2, PRIMES[2]), (c3, PRIMES[3])):
        v = (c * f) * p
        k = (v * (2.0 ** -16)).astype(jnp.int32)
        r = v - k.astype(jnp.float32) * 65536.0
        h = h ^ r.astype(jnp.int32)
    return h


def _sc_body(coords_hbm, tbl_hbm, freq_hbm, enc_hbm,
             freq_v, coords_v, idx_v, rows_v, enc_v, sem_g):
    wid = lax.axis_index("s") * 2 + lax.axis_index("c")
    pltpu.sync_copy(freq_hbm, freq_v)

    lanes = lax.iota(jnp.int32, L)
    iota4 = lanes * 4
    rowpat = lanes // 2
    colpat = lanes % 2
    encpat = colpat * CP + rowpat
    freq_vec = freq_v[...]
    fs = [freq_vec.at[jnp.full_like(lanes, i)].get(mode="promise_in_bounds")
          for i in range(NUM_LEVELS)]

    nchunks = (8 * 65536) // (NW * CP)

    def chunk_body(c, _):
        g = wid * nchunks + c
        pltpu.sync_copy(coords_hbm.at[pl.ds(g * (CP * 4), CP * 4)], coords_v)

        def group_body(q, _):
            cb = q * 64
            cs = [plsc.load_gather(coords_v, [cb + iota4 + j])
                  for j in range(4)]
            cs = [jnp.minimum(jnp.maximum(cj, 0.0), 1.0) for cj in cs]
            for i in range(NUM_LEVELS):
                h = _hash_level(*cs, fs[i])
                idx_v[pl.ds(i * CP + q * 16, 16)] = h + i * TABLE_SIZE
            return 0

        lax.fori_loop(0, GROUPS, group_body, 0, unroll=False)

        def fire(s, _):
            pltpu.async_copy(tbl_hbm.at[idx_v.at[pl.ds(s * 128, 128)]],
                             rows_v.at[pl.ds(s * 128, 128)], sem_g)
            return 0

        lax.fori_loop(0, GSUB, fire, 0, unroll=False)

        def drain(s, _):
            pltpu.make_async_copy(tbl_hbm.at[idx_v.at[pl.ds(s * 128, 128)]],
                                  rows_v.at[pl.ds(s * 128, 128)], sem_g).wait()
            return 0

        lax.fori_loop(0, GSUB, drain, 0, unroll=False)

        def scat_body(g8, _):
            for i in range(NUM_LEVELS):
                val = plsc.load_gather(
                    rows_v, [i * CP + g8 * 8 + rowpat, colpat])
                plsc.store_scatter(
                    enc_v, [2 * i * CP + g8 * 8 + encpat], val)
            return 0

        lax.fori_loop(0, CP // 8, scat_body, 0, unroll=False)

        pltpu.sync_copy(enc_v, enc_hbm.at[pl.ds(g * (CP * D), CP * D)])
        return 0

    lax.fori_loop(0, nchunks, chunk_body, 0, unroll=False)


def _tc_add_body(xt_ref, enc_ref, out_ref):
    out_ref[...] = xt_ref[...] + enc_ref[...].reshape(1, D, CP)


def kernel(x, coords, tables, freq_bands):
    B, N, Dm = x.shape
    P = B * N
    cf = coords.reshape(P * 4)
    tbl = jnp.pad(tables.reshape(NUM_LEVELS * TABLE_SIZE, LEVEL_DIM),
                  ((0, 0), (0, 8 - LEVEL_DIM)))

    mesh = plsc.VectorSubcoreMesh(core_axis_name="c", subcore_axis_name="s")
    sc_run = pl.kernel(
        _sc_body,
        mesh=mesh,
        compiler_params=pltpu.CompilerParams(needs_layout_passes=False,
                                             use_tc_tiling_on_sc=False),
        out_type=jax.ShapeDtypeStruct((P * D,), jnp.float32),
        scratch_types=[
            pltpu.VMEM((NUM_LEVELS,), jnp.float32),
            pltpu.VMEM((CP * 4,), jnp.float32),
            pltpu.VMEM((IDXN,), jnp.int32),
            pltpu.VMEM((IDXN, 8), jnp.float32),
            pltpu.VMEM((CP * D,), jnp.float32),
            pltpu.SemaphoreType.DMA,
        ],
    )
    enc = sc_run(cf, tbl, freq_bands)

    xt = jnp.transpose(x, (0, 2, 1))
    ntile = N // CP
    out_t = pl.pallas_call(
        _tc_add_body,
        grid=(B, ntile),
        in_specs=[
            pl.BlockSpec((1, D, CP), lambda b, t: (b, 0, t)),
            pl.BlockSpec((CP * D,), lambda b, t: (b * ntile + t,)),
        ],
        out_specs=pl.BlockSpec((1, D, CP), lambda b, t: (b, 0, t)),
        out_shape=jax.ShapeDtypeStruct((B, D, N), jnp.float32),
    )(xt, enc)
    return jnp.transpose(out_t, (0, 2, 1))


# speedup vs baseline: 22.0952x; 2.4551x over previous
import jax
import jax.numpy as jnp
from jax import lax
from jax.experimental import pallas as pl
from jax.experimental.pallas import tpu as pltpu
from jax.experimental.pallas import tpu_sc as plsc

NUM_LEVELS = 16
LEVEL_DIM = 2
TABLE_SIZE = 65536
D = NUM_LEVELS * LEVEL_DIM
PRIMES = (1.0, 2654435761.0, 805459861.0, 3674653429.0)

NW = 32
CP = 512
GROUPS = CP // 16
IDXN = CP * NUM_LEVELS
GSUB = IDXN // 128
L = 16
NTILES = NUM_LEVELS * (TABLE_SIZE // 128)
TB = 8
NBATCH = NTILES // TB // 16


def _hash_level(c0, c1, c2, c3, f):
    t0 = c0 * f
    h = t0.astype(jnp.int32)
    for c, p in ((c1, PRIMES[1]), (c2, PRIMES[2]), (c3, PRIMES[3])):
        v = (c * f) * p
        k = (v * (2.0 ** -16)).astype(jnp.int32)
        r = v - k.astype(jnp.float32) * 65536.0
        h = h ^ r.astype(jnp.int32)
    return h


def _sc_body(cf_hbm, tf_hbm, freq_hbm, enc_hbm, tblp_hbm,
             freq_v, coords_v, idx_v, rows_v, enc_v, tbuf_v, slab_v, sem_g):
    cidx = lax.axis_index("c")
    sid = lax.axis_index("s")
    wid = sid * 2 + cidx
    pltpu.sync_copy(freq_hbm, freq_v)

    lanes = lax.iota(jnp.int32, L)
    rowpat = lanes // 2
    colpat = lanes % 2
    encpat = colpat * CP + rowpat

    core_rows = cidx * (NUM_LEVELS * TABLE_SIZE)

    def relay_body(kb, _):
        t0 = (sid * NBATCH + kb) * TB
        pltpu.sync_copy(tf_hbm.at[pl.ds(t0 * 256, TB * 256)], tbuf_v)
        for kt in range(TB):
            for g in range(8):
                v0 = tbuf_v[pl.ds(kt * 256 + g * 16, 16)]
                v1 = tbuf_v[pl.ds(kt * 256 + 128 + g * 16, 16)]
                rows = kt * 128 + g * 16 + lanes
                plsc.store_scatter(slab_v, [rows, lanes * 0], v0)
                plsc.store_scatter(slab_v, [rows, lanes * 0 + 1], v1)
        pltpu.sync_copy(slab_v,
                        tblp_hbm.at[pl.ds(core_rows + t0 * 128, TB * 128), :])
        return 0

    lax.fori_loop(0, NBATCH, relay_body, 0, unroll=False)
    plsc.subcore_barrier()

    freq_vec = freq_v[...]
    fs = [freq_vec.at[jnp.full_like(lanes, i)].get(mode="promise_in_bounds")
          for i in range(NUM_LEVELS)]
    core_base = jnp.full_like(lanes, core_rows)

    nchunks = (8 * 65536) // (NW * CP)

    def chunk_body(c, _):
        g = wid * nchunks + c
        pltpu.sync_copy(cf_hbm.at[pl.ds(g * (CP * 4), CP * 4)], coords_v)

        def group_body(q, _):
            cb = (q // 8) * 512 + (q % 8) * 16
            cs = [coords_v[pl.ds(cb + j * 128, 16)] for j in range(4)]
            cs = [jnp.minimum(jnp.maximum(cj, 0.0), 1.0) for cj in cs]
            for i in range(NUM_LEVELS):
                h = _hash_level(*cs, fs[i])
                idx_v[pl.ds(i * CP + q * 16, 16)] = (
                    h + i * TABLE_SIZE) + core_base
            return 0

        lax.fori_loop(0, GROUPS, group_body, 0, unroll=False)

        def fire(s, _):
            pltpu.async_copy(tblp_hbm.at[idx_v.at[pl.ds(s * 128, 128)]],
                             rows_v.at[pl.ds(s * 128, 128)], sem_g)
            return 0

        lax.fori_loop(0, GSUB, fire, 0, unroll=False)

        def drain(s, _):
            pltpu.make_async_copy(
                tblp_hbm.at[idx_v.at[pl.ds(s * 128, 128)]],
                rows_v.at[pl.ds(s * 128, 128)], sem_g).wait()
            return 0

        lax.fori_loop(0, GSUB, drain, 0, unroll=False)

        def scat_body(g8, _):
            for i in range(NUM_LEVELS):
                val = plsc.load_gather(
                    rows_v, [i * CP + g8 * 8 + rowpat, colpat])
                plsc.store_scatter(
                    enc_v, [2 * i * CP + g8 * 8 + encpat], val)
            return 0

        lax.fori_loop(0, CP // 8, scat_body, 0, unroll=False)

        pltpu.sync_copy(enc_v, enc_hbm.at[pl.ds(g * (CP * D), CP * D)])
        return 0

    lax.fori_loop(0, nchunks, chunk_body, 0, unroll=False)


def _tc_add_body(xt_ref, enc_ref, out_ref):
    out_ref[...] = xt_ref[...] + enc_ref[...].reshape(1, D, CP)


def kernel(x, coords, tables, freq_bands):
    B, N, Dm = x.shape
    P = B * N
    ct = jnp.transpose(coords, (0, 2, 1))
    cf = ct.reshape(B, 4, N // 128, 128).transpose(0, 2, 1, 3).reshape(P * 4)
    tt = jnp.transpose(tables, (0, 2, 1))
    tf = (tt.reshape(NUM_LEVELS, LEVEL_DIM, TABLE_SIZE // 128, 128)
          .transpose(0, 2, 1, 3).reshape(NUM_LEVELS * TABLE_SIZE * LEVEL_DIM))

    mesh = plsc.VectorSubcoreMesh(core_axis_name="c", subcore_axis_name="s")
    sc_run = pl.kernel(
        _sc_body,
        mesh=mesh,
        compiler_params=pltpu.CompilerParams(needs_layout_passes=False,
                                             use_tc_tiling_on_sc=False),
        out_type=(
            jax.ShapeDtypeStruct((P * D,), jnp.float32),
            jax.ShapeDtypeStruct((2 * NUM_LEVELS * TABLE_SIZE, 8),
                                 jnp.float32),
        ),
        scratch_types=[
            pltpu.VMEM((NUM_LEVELS,), jnp.float32),
            pltpu.VMEM((CP * 4,), jnp.float32),
            pltpu.VMEM((IDXN,), jnp.int32),
            pltpu.VMEM((IDXN, 8), jnp.float32),
            pltpu.VMEM((CP * D,), jnp.float32),
            pltpu.VMEM((TB * 256,), jnp.float32),
            pltpu.VMEM((TB * 128, 8), jnp.float32),
            pltpu.SemaphoreType.DMA,
        ],
    )
    enc, _ = sc_run(cf, tf, freq_bands)

    xt = jnp.transpose(x, (0, 2, 1))
    ntile = N // CP
    out_t = pl.pallas_call(
        _tc_add_body,
        grid=(B, ntile),
        in_specs=[
            pl.BlockSpec((1, D, CP), lambda b, t: (b, 0, t)),
            pl.BlockSpec((CP * D,), lambda b, t: (b * ntile + t,)),
        ],
        out_specs=pl.BlockSpec((1, D, CP), lambda b, t: (b, 0, t)),
        out_shape=jax.ShapeDtypeStruct((B, D, N), jnp.float32),
    )(xt, enc)
    return jnp.transpose(out_t, (0, 2, 1))


# speedup vs baseline: 36.1004x; 1.6339x over previous
import jax
import jax.numpy as jnp
from jax import lax
from jax.experimental import pallas as pl
from jax.experimental.pallas import tpu as pltpu
from jax.experimental.pallas import tpu_sc as plsc

NUM_LEVELS = 16
LEVEL_DIM = 2
TABLE_SIZE = 65536
D = NUM_LEVELS * LEVEL_DIM
PRIMES = (1.0, 2654435761.0, 805459861.0, 3674653429.0)

NW = 32
CP = 512
GROUPS = CP // 16
IDXN = CP * NUM_LEVELS
GSUB = IDXN // 128
L = 16
NTILES = NUM_LEVELS * (TABLE_SIZE // 128)
TB = 8
NBATCH = NTILES // TB // 16
XT = 16


def _hash_level(c0, c1, c2, c3, f):
    t0 = c0 * f
    h = t0.astype(jnp.int32)
    for c, p in ((c1, PRIMES[1]), (c2, PRIMES[2]), (c3, PRIMES[3])):
        v = (c * f) * p
        k = (v * (2.0 ** -16)).astype(jnp.int32)
        r = v - k.astype(jnp.float32) * 65536.0
        h = h ^ r.astype(jnp.int32)
    return h


def _sc_body(cf_hbm, tf_hbm, xf_hbm, freq_hbm, out_hbm, tblp_hbm,
             freq_v, coords_v, idx_v, rows_v, out_v, tbuf_v, slab_v,
             sem_g, sem_x):
    cidx = lax.axis_index("c")
    sid = lax.axis_index("s")
    wid = sid * 2 + cidx
    pltpu.sync_copy(freq_hbm, freq_v)

    lanes = lax.iota(jnp.int32, L)
    rowpat = lanes // 2
    colpat = lanes % 2
    xaddpat = colpat * 128 + rowpat

    core_rows = cidx * (NUM_LEVELS * TABLE_SIZE)

    def relay_body(kb, _):
        t0 = (sid * NBATCH + kb) * TB
        pltpu.sync_copy(tf_hbm.at[pl.ds(t0 * 256, TB * 256)], tbuf_v)
        for kt in range(TB):
            for g in range(8):
                v0 = tbuf_v[pl.ds(kt * 256 + g * 16, 16)]
                v1 = tbuf_v[pl.ds(kt * 256 + 128 + g * 16, 16)]
                rows = kt * 128 + g * 16 + lanes
                plsc.store_scatter(slab_v, [rows, lanes * 0], v0)
                plsc.store_scatter(slab_v, [rows, lanes * 0 + 1], v1)
        pltpu.sync_copy(slab_v,
                        tblp_hbm.at[pl.ds(core_rows + t0 * 128, TB * 128), :])
        return 0

    lax.fori_loop(0, NBATCH, relay_body, 0, unroll=False)
    plsc.subcore_barrier()

    freq_vec = freq_v[...]
    fs = [freq_vec.at[jnp.full_like(lanes, i)].get(mode="promise_in_bounds")
          for i in range(NUM_LEVELS)]
    core_base = jnp.full_like(lanes, core_rows)

    nchunks = (8 * 65536) // (NW * CP)

    def chunk_body(c, _):
        g = wid * nchunks + c
        b = g // 128
        nt4 = (g % 128) * 4
        pltpu.sync_copy(cf_hbm.at[pl.ds(g * (CP * 4), CP * 4)], coords_v)

        def xoff(k):
            dt, ntc = k // 4, k % 4
            return (b * 2097152 + dt * 524288 + (nt4 + ntc) * 1024,
                    dt * 4096 + ntc * 1024)

        def xfire(k, _):
            src, dst = xoff(k)
            pltpu.async_copy(xf_hbm.at[pl.ds(src, 1024)],
                             out_v.at[pl.ds(dst, 1024)], sem_x)
            return 0

        lax.fori_loop(0, XT, xfire, 0, unroll=False)

        def group_body(q, _):
            cb = (q // 8) * 512 + (q % 8) * 16
            cs = [coords_v[pl.ds(cb + j * 128, 16)] for j in range(4)]
            cs = [jnp.minimum(jnp.maximum(cj, 0.0), 1.0) for cj in cs]
            for i in range(NUM_LEVELS):
                h = _hash_level(*cs, fs[i])
                idx_v[pl.ds(i * CP + q * 16, 16)] = (
                    h + i * TABLE_SIZE) + core_base
            return 0

        lax.fori_loop(0, GROUPS, group_body, 0, unroll=False)

        def fire(s, _):
            pltpu.async_copy(tblp_hbm.at[idx_v.at[pl.ds(s * 128, 128)]],
                             rows_v.at[pl.ds(s * 128, 128)], sem_g)
            return 0

        lax.fori_loop(0, GSUB, fire, 0, unroll=False)

        def drain(s, _):
            pltpu.make_async_copy(
                tblp_hbm.at[idx_v.at[pl.ds(s * 128, 128)]],
                rows_v.at[pl.ds(s * 128, 128)], sem_g).wait()
            return 0

        lax.fori_loop(0, GSUB, drain, 0, unroll=False)

        def xdrain(k, _):
            src, dst = xoff(k)
            pltpu.make_async_copy(xf_hbm.at[pl.ds(src, 1024)],
                                  out_v.at[pl.ds(dst, 1024)], sem_x).wait()
            return 0

        lax.fori_loop(0, XT, xdrain, 0, unroll=False)

        def scat_body(g8, _):
            nbase = (g8 // 16) * 1024 + (g8 % 16) * 8
            for i in range(NUM_LEVELS):
                val = plsc.load_gather(
                    rows_v, [i * CP + g8 * 8 + rowpat, colpat])
                dbase = (i // 4) * 4096 + ((2 * i) % 8) * 128
                plsc.addupdate_scatter(
                    out_v, [nbase + dbase + xaddpat], val)
            return 0

        lax.fori_loop(0, CP // 8, scat_body, 0, unroll=False)

        def ofire(k, _):
            src, dst = xoff(k)
            pltpu.async_copy(out_v.at[pl.ds(dst, 1024)],
                             out_hbm.at[pl.ds(src, 1024)], sem_x)
            return 0

        lax.fori_loop(0, XT, ofire, 0, unroll=False)

        def odrain(k, _):
            src, dst = xoff(k)
            pltpu.make_async_copy(out_v.at[pl.ds(dst, 1024)],
                                  out_hbm.at[pl.ds(src, 1024)], sem_x).wait()
            return 0

        lax.fori_loop(0, XT, odrain, 0, unroll=False)
        return 0

    lax.fori_loop(0, nchunks, chunk_body, 0, unroll=False)


def kernel(x, coords, tables, freq_bands):
    B, N, Dm = x.shape
    P = B * N
    ct = jnp.transpose(coords, (0, 2, 1))
    cf = ct.reshape(B, 4, N // 128, 128).transpose(0, 2, 1, 3).reshape(P * 4)
    tt = jnp.transpose(tables, (0, 2, 1))
    tf = (tt.reshape(NUM_LEVELS, LEVEL_DIM, TABLE_SIZE // 128, 128)
          .transpose(0, 2, 1, 3).reshape(NUM_LEVELS * TABLE_SIZE * LEVEL_DIM))
    xt = jnp.transpose(x, (0, 2, 1))
    xf = (xt.reshape(B, 4, 8, N // 128, 128)
          .transpose(0, 1, 3, 2, 4).reshape(P * D))

    mesh = plsc.VectorSubcoreMesh(core_axis_name="c", subcore_axis_name="s")
    sc_run = pl.kernel(
        _sc_body,
        mesh=mesh,
        compiler_params=pltpu.CompilerParams(needs_layout_passes=False,
                                             use_tc_tiling_on_sc=False),
        out_type=(
            jax.ShapeDtypeStruct((P * D,), jnp.float32),
            jax.ShapeDtypeStruct((2 * NUM_LEVELS * TABLE_SIZE, 8),
                                 jnp.float32),
        ),
        scratch_types=[
            pltpu.VMEM((NUM_LEVELS,), jnp.float32),
            pltpu.VMEM((CP * 4,), jnp.float32),
            pltpu.VMEM((IDXN,), jnp.int32),
            pltpu.VMEM((IDXN, 8), jnp.float32),
            pltpu.VMEM((CP * D,), jnp.float32),
            pltpu.VMEM((TB * 256,), jnp.float32),
            pltpu.VMEM((TB * 128, 8), jnp.float32),
            pltpu.SemaphoreType.DMA,
            pltpu.SemaphoreType.DMA,
        ],
    )
    of, _ = sc_run(cf, tf, xf, freq_bands)

    ot = (of.reshape(B, 4, N // 128, 8, 128).transpose(0, 1, 3, 2, 4)
          .reshape(B, D, N))
    return jnp.transpose(ot, (0, 2, 1))
